# Initial kernel scaffold; baseline (speedup 1.0000x reference)
import jax
import jax.numpy as jnp
from jax.experimental import pallas as pl


def kernel(x, edge_index, edge_attr, batch, W_emb, b_emb, Wf, bf, Ws, bs, gamma, beta, W_fc, b_fc, W_out, b_out):
    raise NotImplementedError("write your pallas kernel here")


# speedup vs baseline: 1.0326x; 1.0326x over previous
import functools

import jax
import jax.numpy as jnp
from jax.experimental import pallas as pl
from jax.experimental.pallas import tpu as pltpu

N = 50000
E = 800000
ND = 64
ED = 16
G = 256
L = 3

EBLK = 4096


def _msg_body(g_ref, out_ref):
    g = g_ref[...]
    gf = g[:, :ND]
    gs = g[:, ND:]
    sig = 1.0 / (1.0 + jnp.exp(-gf))
    sp = jnp.maximum(gs, 0.0) + jnp.log1p(jnp.exp(-jnp.abs(gs)))
    out_ref[...] = sig * sp


def _msg(g):
    e = g.shape[0]
    grid = (e // EBLK,)
    return pl.pallas_call(
        _msg_body,
        grid=grid,
        in_specs=[pl.BlockSpec((EBLK, 2 * ND), lambda i: (i, 0))],
        out_specs=pl.BlockSpec((EBLK, ND), lambda i: (i, 0)),
        out_shape=jax.ShapeDtypeStruct((e, ND), jnp.float32),
    )(g)


def kernel(x, edge_index, edge_attr, batch, W_emb, b_emb, Wf, bf, Ws, bs, gamma, beta, W_fc, b_fc, W_out, b_out):
    src = edge_index[0]
    dst = edge_index[1]
    h = x @ W_emb + b_emb
    for l in range(L):
        Wi = jnp.concatenate([Wf[l, :ND], Ws[l, :ND]], axis=1)
        Wj = jnp.concatenate([Wf[l, ND:2 * ND], Ws[l, ND:2 * ND]], axis=1)
        We = jnp.concatenate([Wf[l, 2 * ND:], Ws[l, 2 * ND:]], axis=1)
        bb = jnp.concatenate([bf[l], bs[l]])
        P = h @ Wi
        Q = h @ Wj
        C = edge_attr @ We + bb
        g = P[dst] + Q[src] + C
        msg = _msg(g)
        agg = jax.ops.segment_sum(msg, dst, num_segments=N)
        mean = jnp.mean(agg, axis=0)
        var = jnp.var(agg, axis=0)
        agg = (agg - mean) / jnp.sqrt(var + 1e-5) * gamma[l] + beta[l]
        h = h + agg
    sums = jax.ops.segment_sum(h, batch, num_segments=G)
    counts = jax.ops.segment_sum(jnp.ones((N, 1), dtype=h.dtype), batch, num_segments=G)
    pooled = sums / jnp.maximum(counts, 1.0)
    y = jax.nn.softplus(pooled)
    y = y @ W_fc + b_fc
    y = jax.nn.softplus(y)
    y = y @ W_out + b_out
    return y


# speedup vs baseline: 1.8386x; 1.7805x over previous
import functools

import jax
import jax.numpy as jnp
from jax import lax
from jax.experimental import pallas as pl
from jax.experimental.pallas import tpu as pltpu
from jax.experimental.pallas import tpu_sc as plsc

N = 50000
E = 800000
ND = 64
ED = 16
G = 256
L = 3

NC = 2
NS = 16
NW = NC * NS

EBLK = 8192
CH = 128
NCH = E // CH
CPW = (NCH + NW - 1) // NW

_sc_mesh = plsc.VectorSubcoreMesh(core_axis_name="c", subcore_axis_name="s")


@functools.partial(
    pl.kernel,
    out_type=(
        jax.ShapeDtypeStruct((E, 2 * ND), jnp.float32),
        jax.ShapeDtypeStruct((E, 2 * ND), jnp.float32),
    ),
    mesh=_sc_mesh,
    scratch_types=[
        pltpu.VMEM((CH,), jnp.int32),
        pltpu.VMEM((CH,), jnp.int32),
        pltpu.VMEM((CH, 2 * ND), jnp.float32),
        pltpu.VMEM((CH, 2 * ND), jnp.float32),
        pltpu.SemaphoreType.DMA,
        pltpu.SemaphoreType.DMA,
    ],
)
def _sc_gather(p_hbm, q_hbm, dst_hbm, src_hbm, gp_hbm, gq_hbm,
               di, si, pr, qr, s1, s2):
    wid = lax.axis_index("s") * NC + lax.axis_index("c")

    def body(i, carry):
        ci = wid * CPW + i

        @pl.when(ci < NCH)
        def _():
            base = ci * CH
            pltpu.sync_copy(dst_hbm.at[pl.ds(base, CH)], di)
            pltpu.sync_copy(src_hbm.at[pl.ds(base, CH)], si)
            cp = pltpu.async_copy(p_hbm.at[di], pr, s1)
            cq = pltpu.async_copy(q_hbm.at[si], qr, s2)
            cp.wait()
            cq.wait()
            pltpu.sync_copy(pr, gp_hbm.at[pl.ds(base, CH)])
            pltpu.sync_copy(qr, gq_hbm.at[pl.ds(base, CH)])

        return carry

    lax.fori_loop(0, CPW, body, 0)


def _msg_body(gp_ref, gq_ref, ea_ref, we_ref, bb_ref, out_ref):
    c = jnp.dot(ea_ref[...], we_ref[...], preferred_element_type=jnp.float32)
    g = gp_ref[...] + gq_ref[...] + c + bb_ref[...]
    gf = g[:, :ND]
    gs = g[:, ND:]
    sig = 1.0 / (1.0 + jnp.exp(-gf))
    sp = jnp.maximum(gs, 0.0) + jnp.log1p(jnp.exp(-jnp.abs(gs)))
    out_ref[...] = sig * sp


def _msg(gp, gq, ea, We, bb):
    grid = (E // EBLK,)
    return pl.pallas_call(
        _msg_body,
        grid=grid,
        in_specs=[
            pl.BlockSpec((EBLK, 2 * ND), lambda i: (i, 0)),
            pl.BlockSpec((EBLK, 2 * ND), lambda i: (i, 0)),
            pl.BlockSpec((EBLK, ED), lambda i: (i, 0)),
            pl.BlockSpec((ED, 2 * ND), lambda i: (0, 0)),
            pl.BlockSpec((1, 2 * ND), lambda i: (0, 0)),
        ],
        out_specs=pl.BlockSpec((EBLK, ND), lambda i: (i, 0)),
        out_shape=jax.ShapeDtypeStruct((E, ND), jnp.float32),
    )(gp, gq, ea, We, bb)


def kernel(x, edge_index, edge_attr, batch, W_emb, b_emb, Wf, bf, Ws, bs, gamma, beta, W_fc, b_fc, W_out, b_out):
    src = edge_index[0]
    dst = edge_index[1]
    h = x @ W_emb + b_emb
    for l in range(L):
        Wi = jnp.concatenate([Wf[l, :ND], Ws[l, :ND]], axis=1)
        Wj = jnp.concatenate([Wf[l, ND:2 * ND], Ws[l, ND:2 * ND]], axis=1)
        We = jnp.concatenate([Wf[l, 2 * ND:], Ws[l, 2 * ND:]], axis=1)
        bb = jnp.concatenate([bf[l], bs[l]])[None, :]
        P = h @ Wi
        Q = h @ Wj
        gp, gq = _sc_gather(P, Q, dst, src)
        msg = _msg(gp, gq, edge_attr, We, bb)
        agg = jax.ops.segment_sum(msg, dst, num_segments=N)
        mean = jnp.mean(agg, axis=0)
        var = jnp.var(agg, axis=0)
        agg = (agg - mean) / jnp.sqrt(var + 1e-5) * gamma[l] + beta[l]
        h = h + agg
    sums = jax.ops.segment_sum(h, batch, num_segments=G)
    counts = jax.ops.segment_sum(jnp.ones((N, 1), dtype=h.dtype), batch, num_segments=G)
    pooled = sums / jnp.maximum(counts, 1.0)
    y = jax.nn.softplus(pooled)
    y = y @ W_fc + b_fc
    y = jax.nn.softplus(y)
    y = y @ W_out + b_out
    return y
